# Initial kernel scaffold; baseline (speedup 1.0000x reference)
import jax
import jax.numpy as jnp
from jax.experimental import pallas as pl


def kernel(x, pe, source_encoding):
    raise NotImplementedError("write your pallas kernel here")


# speedup vs baseline: 5.3074x; 5.3074x over previous
import jax
import jax.numpy as jnp
from jax.experimental import pallas as pl
from jax.experimental.pallas import tpu as pltpu


def _add_pe_body(s_ref, x_ref, pe_ref, o_ref):
    o_ref[...] = x_ref[...] + pe_ref[...]


def kernel(x, pe, source_encoding):
    var_num, batch, d_model = x.shape
    pe3d = pe.reshape(pe.shape[1], 1, pe.shape[2])
    return pl.pallas_call(
        _add_pe_body,
        grid_spec=pltpu.PrefetchScalarGridSpec(
            num_scalar_prefetch=1,
            grid=(var_num,),
            in_specs=[
                pl.BlockSpec((1, batch, d_model), lambda i, s: (i, 0, 0)),
                pl.BlockSpec((1, 1, d_model), lambda i, s: (s[i], 0, 0)),
            ],
            out_specs=pl.BlockSpec((1, batch, d_model), lambda i, s: (i, 0, 0)),
        ),
        out_shape=jax.ShapeDtypeStruct(x.shape, x.dtype),
    )(source_encoding, x, pe3d)


# speedup vs baseline: 11.5171x; 2.1700x over previous
import jax
import jax.numpy as jnp
from jax.experimental import pallas as pl
from jax.experimental.pallas import tpu as pltpu

_ROWS_PER_STEP = 8


def _add_pe_body(s_ref, x_ref, pe_ref, o_ref):
    i = pl.program_id(0)
    for r in range(_ROWS_PER_STEP):
        row = s_ref[i * _ROWS_PER_STEP + r]
        o_ref[r, :, :] = x_ref[r, :, :] + pe_ref[row, :, :]


def kernel(x, pe, source_encoding):
    var_num, batch, d_model = x.shape
    max_len = pe.shape[1]
    pe3d = pe.reshape(max_len, 1, d_model)
    grid = (var_num // _ROWS_PER_STEP,)
    return pl.pallas_call(
        _add_pe_body,
        grid_spec=pltpu.PrefetchScalarGridSpec(
            num_scalar_prefetch=1,
            grid=grid,
            in_specs=[
                pl.BlockSpec((_ROWS_PER_STEP, batch, d_model),
                             lambda i, s: (i, 0, 0)),
                pl.BlockSpec((max_len, 1, d_model), lambda i, s: (0, 0, 0)),
            ],
            out_specs=pl.BlockSpec((_ROWS_PER_STEP, batch, d_model),
                                   lambda i, s: (i, 0, 0)),
        ),
        out_shape=jax.ShapeDtypeStruct(x.shape, x.dtype),
    )(source_encoding, x, pe3d)


# speedup vs baseline: 11.7903x; 1.0237x over previous
import jax
import jax.numpy as jnp
from jax.experimental import pallas as pl
from jax.experimental.pallas import tpu as pltpu

_ROWS_PER_STEP = 20


def _add_pe_body(s_ref, x_ref, pe_ref, o_ref):
    i = pl.program_id(0)
    for r in range(_ROWS_PER_STEP):
        row = s_ref[i * _ROWS_PER_STEP + r]
        o_ref[r, :, :] = x_ref[r, :, :] + pe_ref[row, :, :]


def kernel(x, pe, source_encoding):
    var_num, batch, d_model = x.shape
    max_len = pe.shape[1]
    pe3d = pe.reshape(max_len, 1, d_model)
    grid = (var_num // _ROWS_PER_STEP,)
    return pl.pallas_call(
        _add_pe_body,
        grid_spec=pltpu.PrefetchScalarGridSpec(
            num_scalar_prefetch=1,
            grid=grid,
            in_specs=[
                pl.BlockSpec((_ROWS_PER_STEP, batch, d_model),
                             lambda i, s: (i, 0, 0)),
                pl.BlockSpec((max_len, 1, d_model), lambda i, s: (0, 0, 0)),
            ],
            out_specs=pl.BlockSpec((_ROWS_PER_STEP, batch, d_model),
                                   lambda i, s: (i, 0, 0)),
        ),
        out_shape=jax.ShapeDtypeStruct(x.shape, x.dtype),
    )(source_encoding, x, pe3d)
